# baseline (speedup 1.0000x reference)
import functools

import jax
import jax.numpy as jnp
from jax import lax
from jax.experimental import pallas as pl
from jax.experimental.pallas import tpu as pltpu
from jax.experimental.pallas import tpu_sc as plsc

N = 10000
NC, NS = 2, 16
NW = NC * NS
CHUNK = 128
N_PAD = 10240
RPT = N_PAD // NS
RC = RPT // CHUNK


HMAX = 56
PH0 = (56, 24)
PH1 = (56, 22)
SPLIT0 = sum(PH0)
SPLIT1 = sum(PH1)


def _spmm_sc(D):
  mesh = plsc.VectorSubcoreMesh(core_axis_name="c", subcore_axis_name="s")

  @functools.partial(
      pl.kernel,
      out_type=jax.ShapeDtypeStruct((NC, N_PAD, D), jnp.float32),
      mesh=mesh,
      scratch_types=[
          pltpu.VMEM((2, HMAX, CHUNK), jnp.int32),
          pltpu.VMEM((CHUNK, D), jnp.float32),
          pltpu.VMEM((CHUNK, D), jnp.float32),
          pltpu.VMEM_SHARED((N_PAD, D), jnp.float32),
          pltpu.SemaphoreType.DMA,
          pltpu.SemaphoreType.DMA,
      ],
  )
  def spmm(table, edges, zeros, out, idx, buf0, buf1, acc, sem0, sem1):
    c = lax.axis_index("c")
    s = lax.axis_index("s")
    rbase = s * RPT
    w = c * NS + s

    pltpu.sync_copy(zeros, acc.at[pl.ds(rbase, RPT)])
    plsc.subcore_barrier()

    def run_phase(off, nhalf):
      pltpu.sync_copy(edges.at[:, w, pl.ds(off, HMAX)], idx)
      pltpu.async_copy(table.at[idx.at[0, 0]], buf0, sem0)
      pltpu.async_copy(table.at[idx.at[0, 1]], buf1, sem1)

      def body(j, carry):
        i = 2 * j
        pltpu.make_async_copy(table.at[idx.at[0, i]], buf0, sem0).wait()
        pltpu.sync_copy(buf0, acc.at[idx.at[1, i]], add=True)
        pltpu.async_copy(table.at[idx.at[0, i + 2]], buf0, sem0)
        pltpu.make_async_copy(table.at[idx.at[0, i + 1]], buf1, sem1).wait()
        pltpu.sync_copy(buf1, acc.at[idx.at[1, i + 1]], add=True)
        pltpu.async_copy(table.at[idx.at[0, i + 3]], buf1, sem1)
        return carry

      lax.fori_loop(0, nhalf - 1, body, 0)

      last = 2 * (nhalf - 1)
      pltpu.make_async_copy(table.at[idx.at[0, last]], buf0, sem0).wait()
      pltpu.sync_copy(buf0, acc.at[idx.at[1, last]], add=True)
      pltpu.make_async_copy(table.at[idx.at[0, last + 1]], buf1, sem1).wait()
      pltpu.sync_copy(buf1, acc.at[idx.at[1, last + 1]], add=True)

    run_phase(0, jnp.where(c == 0, PH0[0] // 2, PH1[0] // 2))
    run_phase(HMAX, jnp.where(c == 0, PH0[1] // 2, PH1[1] // 2))

    plsc.subcore_barrier()
    pltpu.sync_copy(acc.at[pl.ds(rbase, RPT)],
                    out.at[c, pl.ds(rbase, RPT)])

  return spmm


def _mm_body(x_ref, w_ref, o_ref):
  o_ref[...] = jnp.dot(x_ref[...], w_ref[...],
                       preferred_element_type=jnp.float32)


def _matmul(x, w):
  m, k = x.shape
  d = w.shape[1]
  bm = m // 4
  return pl.pallas_call(
      _mm_body,
      grid=(4,),
      in_specs=[pl.BlockSpec((bm, k), lambda i: (i, 0)),
                pl.BlockSpec((k, d), lambda i: (0, 0))],
      out_specs=pl.BlockSpec((bm, d), lambda i: (i, 0)),
      out_shape=jax.ShapeDtypeStruct((m, d), jnp.float32),
  )(x, w)


def _fuse_body(relu, pa_ref, pb_ref, b_ref, w_ref, o_ref):
  h = pa_ref[...] + pb_ref[...] + b_ref[...]
  if relu:
    h = jnp.maximum(h, 0.0)
  o_ref[...] = jnp.dot(h, w_ref[...], preferred_element_type=jnp.float32)


def _fuse_matmul(pa, pb, b, w, relu):
  m, k = pa.shape
  d = w.shape[1]
  bm = m // 4
  return pl.pallas_call(
      functools.partial(_fuse_body, relu),
      grid=(4,),
      in_specs=[pl.BlockSpec((bm, k), lambda i: (i, 0)),
                pl.BlockSpec((bm, k), lambda i: (i, 0)),
                pl.BlockSpec((1, k), lambda i: (0, 0)),
                pl.BlockSpec((k, d), lambda i: (0, 0))],
      out_specs=pl.BlockSpec((bm, d), lambda i: (i, 0)),
      out_shape=jax.ShapeDtypeStruct((m, d), jnp.float32),
  )(pa, pb, b.reshape(1, k), w)


def _final_body(pa_ref, pb_ref, b_ref, o_ref):
  h = pa_ref[...] + pb_ref[...] + b_ref[...]
  m = jnp.max(h, axis=1, keepdims=True)
  lse = jnp.log(jnp.sum(jnp.exp(h - m), axis=1, keepdims=True)) + m
  o_ref[...] = h - lse


def _final(pa, pb, b):
  m, d = pa.shape
  bm = m // 4
  return pl.pallas_call(
      _final_body,
      grid=(4,),
      in_specs=[pl.BlockSpec((bm, d), lambda i: (i, 0)),
                pl.BlockSpec((bm, d), lambda i: (i, 0)),
                pl.BlockSpec((1, d), lambda i: (0, 0))],
      out_specs=pl.BlockSpec((bm, d), lambda i: (i, 0)),
      out_shape=jax.ShapeDtypeStruct((m, d), jnp.float32),
  )(pa, pb, b.reshape(1, d))


def kernel(x, adj, W1, b1, W2, b2, W3, b3):
  e = adj.shape[1]
  tot = NS * (SPLIT0 + SPLIT1)
  assert tot * CHUNK >= e
  e_pad = tot * CHUNK
  adj = adj.astype(jnp.int32)
  src = jnp.concatenate([adj[0], jnp.zeros((e_pad - e,), jnp.int32)])
  pad_dst = N + jnp.arange(e_pad - e, dtype=jnp.int32) % (N_PAD - N)
  dst = jnp.concatenate([adj[1], pad_dst])
  c0 = NS * SPLIT0 * CHUNK
  def per_tile(v):
    a = v[:c0].reshape(NS, SPLIT0, CHUNK)
    b = v[c0:].reshape(NS, SPLIT1, CHUNK)
    z = lambda r: jnp.zeros((NS, r, CHUNK), jnp.int32)
    a = jnp.concatenate([a[:, :PH0[0]], z(HMAX - PH0[0]),
                         a[:, PH0[0]:], z(HMAX - PH0[1])], axis=1)
    b = jnp.concatenate([b[:, :PH1[0]], z(HMAX - PH1[0]),
                         b[:, PH1[0]:], z(HMAX - PH1[1])], axis=1)
    return jnp.concatenate([a, b], axis=0)
  edges = jnp.stack([per_tile(src), per_tile(dst)], axis=0)
  x_pad = jnp.pad(x, ((0, N_PAD - N), (0, 0)))
  z128 = jnp.zeros((RPT, 128), jnp.float32)
  spmm128 = _spmm_sc(128)
  W3p = jnp.pad(W3, ((0, 0), (0, 128 - W3.shape[1])))

  t1 = _matmul(x_pad, W1)
  p1 = spmm128(t1, edges, z128)
  t2 = _fuse_matmul(p1[0], p1[1], b1, W2, relu=True)
  p2 = spmm128(t2, edges, z128)
  t3 = _fuse_matmul(p2[0], p2[1], b2, W3p, relu=False)
  p3 = spmm128(t3, edges, z128)
  out = _final(p3[0, :, :64], p3[1, :, :64], b3)
  return out[:N]


# speedup vs baseline: 1.0445x; 1.0445x over previous
import functools

import jax
import jax.numpy as jnp
from jax import lax
from jax.experimental import pallas as pl
from jax.experimental.pallas import tpu as pltpu
from jax.experimental.pallas import tpu_sc as plsc

N = 10000
NC, NS = 2, 16
NW = NC * NS
CHUNK = 128
N_PAD = 10240
RPT = N_PAD // NS
RC = RPT // CHUNK


HMAX = 56
PH0 = (56, 36)
PH1 = (56, 10)
SPLIT0 = sum(PH0)
SPLIT1 = sum(PH1)


def _spmm_sc(D):
  mesh = plsc.VectorSubcoreMesh(core_axis_name="c", subcore_axis_name="s")

  @functools.partial(
      pl.kernel,
      out_type=jax.ShapeDtypeStruct((NC, N_PAD, D), jnp.float32),
      mesh=mesh,
      scratch_types=[
          pltpu.VMEM((2, HMAX, CHUNK), jnp.int32),
          pltpu.VMEM((CHUNK, D), jnp.float32),
          pltpu.VMEM((CHUNK, D), jnp.float32),
          pltpu.VMEM_SHARED((N_PAD, D), jnp.float32),
          pltpu.SemaphoreType.DMA,
          pltpu.SemaphoreType.DMA,
      ],
  )
  def spmm(table, edges, zeros, out, idx, buf0, buf1, acc, sem0, sem1):
    c = lax.axis_index("c")
    s = lax.axis_index("s")
    rbase = s * RPT
    w = c * NS + s

    pltpu.sync_copy(zeros, acc.at[pl.ds(rbase, RPT)])
    plsc.subcore_barrier()

    def run_phase(off, nhalf):
      pltpu.sync_copy(edges.at[:, w, pl.ds(off, HMAX)], idx)
      pltpu.async_copy(table.at[idx.at[0, 0]], buf0, sem0)
      pltpu.async_copy(table.at[idx.at[0, 1]], buf1, sem1)

      def body(j, carry):
        i = 2 * j
        pltpu.make_async_copy(table.at[idx.at[0, i]], buf0, sem0).wait()
        pltpu.sync_copy(buf0, acc.at[idx.at[1, i]], add=True)
        pltpu.async_copy(table.at[idx.at[0, i + 2]], buf0, sem0)
        pltpu.make_async_copy(table.at[idx.at[0, i + 1]], buf1, sem1).wait()
        pltpu.sync_copy(buf1, acc.at[idx.at[1, i + 1]], add=True)
        pltpu.async_copy(table.at[idx.at[0, i + 3]], buf1, sem1)
        return carry

      lax.fori_loop(0, nhalf - 1, body, 0)

      last = 2 * (nhalf - 1)
      pltpu.make_async_copy(table.at[idx.at[0, last]], buf0, sem0).wait()
      pltpu.sync_copy(buf0, acc.at[idx.at[1, last]], add=True)
      pltpu.make_async_copy(table.at[idx.at[0, last + 1]], buf1, sem1).wait()
      pltpu.sync_copy(buf1, acc.at[idx.at[1, last + 1]], add=True)

    run_phase(0, jnp.where(c == 0, PH0[0] // 2, PH1[0] // 2))
    run_phase(HMAX, jnp.where(c == 0, PH0[1] // 2, PH1[1] // 2))

    plsc.subcore_barrier()
    pltpu.sync_copy(acc.at[pl.ds(rbase, RPT)],
                    out.at[c, pl.ds(rbase, RPT)])

  return spmm


def _mm_body(x_ref, w_ref, o_ref):
  o_ref[...] = jnp.dot(x_ref[...], w_ref[...],
                       preferred_element_type=jnp.float32)


def _matmul(x, w):
  m, k = x.shape
  d = w.shape[1]
  bm = m // 4
  return pl.pallas_call(
      _mm_body,
      grid=(4,),
      in_specs=[pl.BlockSpec((bm, k), lambda i: (i, 0)),
                pl.BlockSpec((k, d), lambda i: (0, 0))],
      out_specs=pl.BlockSpec((bm, d), lambda i: (i, 0)),
      out_shape=jax.ShapeDtypeStruct((m, d), jnp.float32),
  )(x, w)


def _fuse_body(relu, pa_ref, pb_ref, b_ref, w_ref, o_ref):
  h = pa_ref[...] + pb_ref[...] + b_ref[...]
  if relu:
    h = jnp.maximum(h, 0.0)
  o_ref[...] = jnp.dot(h, w_ref[...], preferred_element_type=jnp.float32)


def _fuse_matmul(pa, pb, b, w, relu):
  m, k = pa.shape
  d = w.shape[1]
  bm = m // 4
  return pl.pallas_call(
      functools.partial(_fuse_body, relu),
      grid=(4,),
      in_specs=[pl.BlockSpec((bm, k), lambda i: (i, 0)),
                pl.BlockSpec((bm, k), lambda i: (i, 0)),
                pl.BlockSpec((1, k), lambda i: (0, 0)),
                pl.BlockSpec((k, d), lambda i: (0, 0))],
      out_specs=pl.BlockSpec((bm, d), lambda i: (i, 0)),
      out_shape=jax.ShapeDtypeStruct((m, d), jnp.float32),
  )(pa, pb, b.reshape(1, k), w)


def _final_body(pa_ref, pb_ref, b_ref, o_ref):
  h = pa_ref[...] + pb_ref[...] + b_ref[...]
  m = jnp.max(h, axis=1, keepdims=True)
  lse = jnp.log(jnp.sum(jnp.exp(h - m), axis=1, keepdims=True)) + m
  o_ref[...] = h - lse


def _final(pa, pb, b):
  m, d = pa.shape
  bm = m // 4
  return pl.pallas_call(
      _final_body,
      grid=(4,),
      in_specs=[pl.BlockSpec((bm, d), lambda i: (i, 0)),
                pl.BlockSpec((bm, d), lambda i: (i, 0)),
                pl.BlockSpec((1, d), lambda i: (0, 0))],
      out_specs=pl.BlockSpec((bm, d), lambda i: (i, 0)),
      out_shape=jax.ShapeDtypeStruct((m, d), jnp.float32),
  )(pa, pb, b.reshape(1, d))


def kernel(x, adj, W1, b1, W2, b2, W3, b3):
  e = adj.shape[1]
  tot = NS * (SPLIT0 + SPLIT1)
  assert tot * CHUNK >= e
  e_pad = tot * CHUNK
  adj = adj.astype(jnp.int32)
  src = jnp.concatenate([adj[0], jnp.zeros((e_pad - e,), jnp.int32)])
  pad_dst = N + jnp.arange(e_pad - e, dtype=jnp.int32) % (N_PAD - N)
  dst = jnp.concatenate([adj[1], pad_dst])
  c0 = NS * SPLIT0 * CHUNK
  def per_tile(v):
    a = v[:c0].reshape(NS, SPLIT0, CHUNK)
    b = v[c0:].reshape(NS, SPLIT1, CHUNK)
    z = lambda r: jnp.zeros((NS, r, CHUNK), jnp.int32)
    a = jnp.concatenate([a[:, :PH0[0]], z(HMAX - PH0[0]),
                         a[:, PH0[0]:], z(HMAX - PH0[1])], axis=1)
    b = jnp.concatenate([b[:, :PH1[0]], z(HMAX - PH1[0]),
                         b[:, PH1[0]:], z(HMAX - PH1[1])], axis=1)
    return jnp.concatenate([a, b], axis=0)
  edges = jnp.stack([per_tile(src), per_tile(dst)], axis=0)
  x_pad = jnp.pad(x, ((0, N_PAD - N), (0, 0)))
  z128 = jnp.zeros((RPT, 128), jnp.float32)
  spmm128 = _spmm_sc(128)
  W3p = jnp.pad(W3, ((0, 0), (0, 128 - W3.shape[1])))

  t1 = _matmul(x_pad, W1)
  p1 = spmm128(t1, edges, z128)
  t2 = _fuse_matmul(p1[0], p1[1], b1, W2, relu=True)
  p2 = spmm128(t2, edges, z128)
  t3 = _fuse_matmul(p2[0], p2[1], b2, W3p, relu=False)
  p3 = spmm128(t3, edges, z128)
  out = _final(p3[0, :, :64], p3[1, :, :64], b3)
  return out[:N]


# speedup vs baseline: 1.0603x; 1.0151x over previous
import functools

import jax
import jax.numpy as jnp
from jax import lax
from jax.experimental import pallas as pl
from jax.experimental.pallas import tpu as pltpu
from jax.experimental.pallas import tpu_sc as plsc

N = 10000
NC, NS = 2, 16
NW = NC * NS
CHUNK = 128
N_PAD = 10240
RPT = N_PAD // NS
RC = RPT // CHUNK


HMAX = 56
PH0 = (56, 56)
PH1 = (24, 22)
SPLIT0 = sum(PH0)
SPLIT1 = sum(PH1)


def _spmm_sc(D):
  mesh = plsc.VectorSubcoreMesh(core_axis_name="c", subcore_axis_name="s")

  @functools.partial(
      pl.kernel,
      out_type=jax.ShapeDtypeStruct((NC, N_PAD, D), jnp.float32),
      mesh=mesh,
      scratch_types=[
          pltpu.VMEM((2, HMAX, CHUNK), jnp.int32),
          pltpu.VMEM((CHUNK, D), jnp.float32),
          pltpu.VMEM((CHUNK, D), jnp.float32),
          pltpu.VMEM_SHARED((N_PAD, D), jnp.float32),
          pltpu.SemaphoreType.DMA,
          pltpu.SemaphoreType.DMA,
      ],
  )
  def spmm(table, edges, zeros, out, idx, buf0, buf1, acc, sem0, sem1):
    c = lax.axis_index("c")
    s = lax.axis_index("s")
    rbase = s * RPT
    w = c * NS + s

    pltpu.sync_copy(zeros, acc.at[pl.ds(rbase, RPT)])
    plsc.subcore_barrier()

    def run_phase(off, nhalf):
      pltpu.sync_copy(edges.at[:, w, pl.ds(off, HMAX)], idx)
      pltpu.async_copy(table.at[idx.at[0, 0]], buf0, sem0)
      pltpu.async_copy(table.at[idx.at[0, 1]], buf1, sem1)

      def body(j, carry):
        i = 2 * j
        pltpu.make_async_copy(table.at[idx.at[0, i]], buf0, sem0).wait()
        pltpu.sync_copy(buf0, acc.at[idx.at[1, i]], add=True)
        pltpu.async_copy(table.at[idx.at[0, i + 2]], buf0, sem0)
        pltpu.make_async_copy(table.at[idx.at[0, i + 1]], buf1, sem1).wait()
        pltpu.sync_copy(buf1, acc.at[idx.at[1, i + 1]], add=True)
        pltpu.async_copy(table.at[idx.at[0, i + 3]], buf1, sem1)
        return carry

      lax.fori_loop(0, nhalf - 1, body, 0)

      last = 2 * (nhalf - 1)
      pltpu.make_async_copy(table.at[idx.at[0, last]], buf0, sem0).wait()
      pltpu.sync_copy(buf0, acc.at[idx.at[1, last]], add=True)
      pltpu.make_async_copy(table.at[idx.at[0, last + 1]], buf1, sem1).wait()
      pltpu.sync_copy(buf1, acc.at[idx.at[1, last + 1]], add=True)

    run_phase(0, jnp.where(c == 0, PH0[0] // 2, PH1[0] // 2))
    run_phase(HMAX, jnp.where(c == 0, PH0[1] // 2, PH1[1] // 2))

    plsc.subcore_barrier()
    pltpu.sync_copy(acc.at[pl.ds(rbase, RPT)],
                    out.at[c, pl.ds(rbase, RPT)])

  return spmm


def _mm_body(x_ref, w_ref, o_ref):
  o_ref[...] = jnp.dot(x_ref[...], w_ref[...],
                       preferred_element_type=jnp.float32)


def _matmul(x, w):
  m, k = x.shape
  d = w.shape[1]
  bm = m // 4
  return pl.pallas_call(
      _mm_body,
      grid=(4,),
      in_specs=[pl.BlockSpec((bm, k), lambda i: (i, 0)),
                pl.BlockSpec((k, d), lambda i: (0, 0))],
      out_specs=pl.BlockSpec((bm, d), lambda i: (i, 0)),
      out_shape=jax.ShapeDtypeStruct((m, d), jnp.float32),
  )(x, w)


def _fuse_body(relu, pa_ref, pb_ref, b_ref, w_ref, o_ref):
  h = pa_ref[...] + pb_ref[...] + b_ref[...]
  if relu:
    h = jnp.maximum(h, 0.0)
  o_ref[...] = jnp.dot(h, w_ref[...], preferred_element_type=jnp.float32)


def _fuse_matmul(pa, pb, b, w, relu):
  m, k = pa.shape
  d = w.shape[1]
  bm = m // 4
  return pl.pallas_call(
      functools.partial(_fuse_body, relu),
      grid=(4,),
      in_specs=[pl.BlockSpec((bm, k), lambda i: (i, 0)),
                pl.BlockSpec((bm, k), lambda i: (i, 0)),
                pl.BlockSpec((1, k), lambda i: (0, 0)),
                pl.BlockSpec((k, d), lambda i: (0, 0))],
      out_specs=pl.BlockSpec((bm, d), lambda i: (i, 0)),
      out_shape=jax.ShapeDtypeStruct((m, d), jnp.float32),
  )(pa, pb, b.reshape(1, k), w)


def _final_body(pa_ref, pb_ref, b_ref, o_ref):
  h = pa_ref[...] + pb_ref[...] + b_ref[...]
  m = jnp.max(h, axis=1, keepdims=True)
  lse = jnp.log(jnp.sum(jnp.exp(h - m), axis=1, keepdims=True)) + m
  o_ref[...] = h - lse


def _final(pa, pb, b):
  m, d = pa.shape
  bm = m // 4
  return pl.pallas_call(
      _final_body,
      grid=(4,),
      in_specs=[pl.BlockSpec((bm, d), lambda i: (i, 0)),
                pl.BlockSpec((bm, d), lambda i: (i, 0)),
                pl.BlockSpec((1, d), lambda i: (0, 0))],
      out_specs=pl.BlockSpec((bm, d), lambda i: (i, 0)),
      out_shape=jax.ShapeDtypeStruct((m, d), jnp.float32),
  )(pa, pb, b.reshape(1, d))


def kernel(x, adj, W1, b1, W2, b2, W3, b3):
  e = adj.shape[1]
  tot = NS * (SPLIT0 + SPLIT1)
  assert tot * CHUNK >= e
  e_pad = tot * CHUNK
  adj = adj.astype(jnp.int32)
  src = jnp.concatenate([adj[0], jnp.zeros((e_pad - e,), jnp.int32)])
  pad_dst = N + jnp.arange(e_pad - e, dtype=jnp.int32) % (N_PAD - N)
  dst = jnp.concatenate([adj[1], pad_dst])
  c0 = NS * SPLIT0 * CHUNK
  def per_tile(v):
    a = v[:c0].reshape(NS, SPLIT0, CHUNK)
    b = v[c0:].reshape(NS, SPLIT1, CHUNK)
    z = lambda r: jnp.zeros((NS, r, CHUNK), jnp.int32)
    a = jnp.concatenate([a[:, :PH0[0]], z(HMAX - PH0[0]),
                         a[:, PH0[0]:], z(HMAX - PH0[1])], axis=1)
    b = jnp.concatenate([b[:, :PH1[0]], z(HMAX - PH1[0]),
                         b[:, PH1[0]:], z(HMAX - PH1[1])], axis=1)
    return jnp.concatenate([a, b], axis=0)
  edges = jnp.stack([per_tile(src), per_tile(dst)], axis=0)
  x_pad = jnp.pad(x, ((0, N_PAD - N), (0, 0)))
  z128 = jnp.zeros((RPT, 128), jnp.float32)
  spmm128 = _spmm_sc(128)
  W3p = jnp.pad(W3, ((0, 0), (0, 128 - W3.shape[1])))

  t1 = _matmul(x_pad, W1)
  p1 = spmm128(t1, edges, z128)
  t2 = _fuse_matmul(p1[0], p1[1], b1, W2, relu=True)
  p2 = spmm128(t2, edges, z128)
  t3 = _fuse_matmul(p2[0], p2[1], b2, W3p, relu=False)
  p3 = spmm128(t3, edges, z128)
  out = _final(p3[0, :, :64], p3[1, :, :64], b3)
  return out[:N]
